# baseline (speedup 1.0000x reference)
import jax
import jax.numpy as jnp
from jax import lax
from jax.experimental import pallas as pl
from jax.experimental.pallas import tpu as pltpu
from jax.sharding import PartitionSpec as P

_BATCH = 32
_VOCAB = 1_000_000
_BLOCK_V = 16384
_GRID = (_VOCAB + _BLOCK_V - 1) // _BLOCK_V
_PADV = _GRID * _BLOCK_V
_CHUNK = 16384
_NCH = _BLOCK_V // _CHUNK
_NSUB = _CHUNK // 128

_KS1 = 42
_KS2 = 0x1BD11BDA ^ 42
_C1 = _KS2 + 1
_C2 = 2
_C3 = _KS1 + 3
_C4 = _KS2 + 4
_C5 = 5
_R0 = (13, 15, 26, 6)
_R1 = (17, 29, 16, 24)


def _rotl(x, d):
    return (x << jnp.uint32(d)) | (x >> jnp.uint32(32 - d))


def _rounds(x0, x1, rots):
    for r in rots:
        x0 = x0 + x1
        x1 = x0 ^ _rotl(x1, r)
    return x0, x1


def _bits_from_x1(x1):
    x0 = x1
    x1 = x0 ^ _rotl(x1, 13)
    x0, x1 = _rounds(x0, x1, _R0[1:])
    x0, x1 = x0 + jnp.uint32(_KS1), x1 + jnp.uint32(_C1)
    x0, x1 = _rounds(x0, x1, _R1)
    x0, x1 = x0 + jnp.uint32(_KS2), x1 + jnp.uint32(_C2)
    x0, x1 = _rounds(x0, x1, _R0)
    x0, x1 = x0, x1 + jnp.uint32(_C3)
    x0, x1 = _rounds(x0, x1, _R1)
    x0, x1 = x0 + jnp.uint32(_KS1), x1 + jnp.uint32(_C4)
    x0, x1 = _rounds(x0, x1, _R0)
    x0, x1 = x0 + jnp.uint32(_KS2), x1 + jnp.uint32(_C5)
    return x0 ^ x1


def _gumbel_from_x1(x1):
    bits = _bits_from_x1(x1)
    float_bits = (bits >> jnp.uint32(9)) | jnp.uint32(0x3F800000)
    f = lax.bitcast_convert_type(float_bits, jnp.float32) - jnp.float32(1.0)
    u = jnp.maximum(f, jnp.float32(1e-20))
    return -jnp.log(-jnp.log(u))


def _make_body(rows):
    def _body(base_ref, logits_ref, mask_ref, out_ref, ids_ref,
              accv_ref, acci_ref):
        j = pl.program_id(0)

        @pl.when(j == 0)
        def _init():
            accv_ref[...] = jnp.full((rows, 128), -jnp.inf, jnp.float32)
            acci_ref[...] = jnp.zeros((rows, 128), jnp.int32)

        colb = jax.lax.broadcasted_iota(jnp.int32, (rows, _CHUNK), 1)
        rowb = jax.lax.broadcasted_iota(jnp.int32, (rows, _CHUNK), 0)
        linbase = ((rowb + base_ref[0]) * _VOCAB + colb
                   + j * _BLOCK_V).astype(jnp.uint32)

        def step(k, carry):
            accv, acci = carry
            off = k * _CHUNK
            lin = linbase + off.astype(jnp.uint32)
            fl = logits_ref[:, pl.ds(off, _CHUNK)] * jnp.float32(2.0)
            fl = fl + mask_ref[:, pl.ds(off, _CHUNK)]
            out_ref[:, pl.ds(off, _CHUNK)] = fl
            cand = fl + _gumbel_from_x1(lin + jnp.uint32(_KS1))
            lin_i = lin.astype(jnp.int32)
            for s in range(_NSUB):
                c = cand[:, s * 128:(s + 1) * 128]
                li = lin_i[:, s * 128:(s + 1) * 128]
                better = c > accv
                acci = jnp.where(better, li, acci)
                accv = jnp.where(better, c, accv)
            return (accv, acci)

        accv, acci = lax.fori_loop(
            0, _NCH, step, (accv_ref[...], acci_ref[...])
        )
        accv_ref[...] = accv
        acci_ref[...] = acci

        @pl.when(j == _GRID - 1)
        def _done():
            row = jax.lax.broadcasted_iota(jnp.int32, (rows, 128), 0)
            col = acci - (row + base_ref[0]) * _VOCAB
            m = jnp.max(accv, axis=1, keepdims=True)
            ids_ref[...] = jnp.min(
                jnp.where(accv == m, col, jnp.int32(2**30)),
                axis=1, keepdims=True,
            )

    return _body


def _run_shard(rows, base, logits_shard, mask2d):
    final_logits, ids = pl.pallas_call(
        _make_body(rows),
        grid_spec=pltpu.PrefetchScalarGridSpec(
            num_scalar_prefetch=1,
            grid=(_GRID,),
            in_specs=[
                pl.BlockSpec((rows, _BLOCK_V), lambda j, s: (0, j)),
                pl.BlockSpec((1, _BLOCK_V), lambda j, s: (0, j)),
            ],
            out_specs=[
                pl.BlockSpec((rows, _BLOCK_V), lambda j, s: (0, j)),
                pl.BlockSpec((rows, 1), lambda j, s: (0, 0)),
            ],
            scratch_shapes=[
                pltpu.VMEM((rows, 128), jnp.float32),
                pltpu.VMEM((rows, 128), jnp.int32),
            ],
        ),
        out_shape=[
            jax.ShapeDtypeStruct((rows, _VOCAB), jnp.float32),
            jax.ShapeDtypeStruct((rows, 1), jnp.int32),
        ],
        compiler_params=pltpu.CompilerParams(
            dimension_semantics=("arbitrary",),
        ),
    )(base, logits_shard, mask2d)
    return final_logits, ids.reshape(rows)


def kernel(logits, prediction_mask):
    mask2d = jnp.concatenate(
        [
            prediction_mask.reshape(1, _VOCAB),
            jnp.full((1, _PADV - _VOCAB), -jnp.inf, jnp.float32),
        ],
        axis=1,
    )
    devs = jax.devices()
    ndev = 2 if len(devs) >= 2 else 1
    rows = _BATCH // ndev
    if ndev == 1:
        base = jnp.zeros((1,), jnp.int32)
        return _run_shard(rows, base, logits, mask2d)

    mesh = jax.make_mesh(
        (ndev,), ("b",),
        axis_types=(jax.sharding.AxisType.Explicit,),
        devices=devs[:ndev],
    )
    logits_s = jax.reshard(
        logits, jax.sharding.NamedSharding(mesh, P("b", None))
    )
    mask_s = jax.reshard(
        mask2d, jax.sharding.NamedSharding(mesh, P(None, None))
    )

    def shard_fn(lg, mk):
        base = (jax.lax.axis_index("b") * rows).reshape(1).astype(jnp.int32)
        return _run_shard(rows, base, lg, mk)

    f = jax.shard_map(
        shard_fn,
        mesh=mesh,
        in_specs=(P("b", None), P(None, None)),
        out_specs=(P("b", None), P("b")),
        check_vma=False,
    )
    return f(logits_s, mask_s)


# speedup vs baseline: 1.0397x; 1.0397x over previous
import jax
import jax.numpy as jnp
from jax import lax
from jax.experimental import pallas as pl
from jax.experimental.pallas import tpu as pltpu
from jax.sharding import PartitionSpec as P

_BATCH = 32
_VOCAB = 1_000_000
_BLOCK_V = 16384
_GRID = (_VOCAB + _BLOCK_V - 1) // _BLOCK_V
_PADV = _GRID * _BLOCK_V
_CHUNK = 16384
_NCH = _BLOCK_V // _CHUNK
_NSUB = _CHUNK // 128

_KS1 = 42
_KS2 = 0x1BD11BDA ^ 42
_C1 = _KS2 + 1
_C2 = 2
_C3 = _KS1 + 3
_C4 = _KS2 + 4
_C5 = 5
_R0 = (13, 15, 26, 6)
_R1 = (17, 29, 16, 24)


def _rotl(x, d):
    return (x << jnp.uint32(d)) | (x >> jnp.uint32(32 - d))


def _rounds(x0, x1, rots):
    for r in rots:
        x0 = x0 + x1
        x1 = x0 ^ _rotl(x1, r)
    return x0, x1


def _bits_from_x1(x1):
    x0 = x1
    x1 = x0 ^ _rotl(x1, 13)
    x0, x1 = _rounds(x0, x1, _R0[1:])
    x0, x1 = x0 + jnp.uint32(_KS1), x1 + jnp.uint32(_C1)
    x0, x1 = _rounds(x0, x1, _R1)
    x0, x1 = x0 + jnp.uint32(_KS2), x1 + jnp.uint32(_C2)
    x0, x1 = _rounds(x0, x1, _R0)
    x0, x1 = x0, x1 + jnp.uint32(_C3)
    x0, x1 = _rounds(x0, x1, _R1)
    x0, x1 = x0 + jnp.uint32(_KS1), x1 + jnp.uint32(_C4)
    x0, x1 = _rounds(x0, x1, _R0)
    x0, x1 = x0 + jnp.uint32(_KS2), x1 + jnp.uint32(_C5)
    return x0 ^ x1


def _gumbel_from_x1(x1):
    bits = _bits_from_x1(x1)
    float_bits = (bits >> jnp.uint32(9)) | jnp.uint32(0x3F800000)
    f = lax.bitcast_convert_type(float_bits, jnp.float32) - jnp.float32(1.0)
    u = jnp.maximum(f, jnp.float32(1e-20))
    return -jnp.log(-jnp.log(u))


def _make_body(rows):
    def _body(base_ref, logits_ref, mask_ref, out_ref, ids_ref,
              accv_ref, acci_ref):
        j = pl.program_id(0)

        @pl.when(j == 0)
        def _init():
            accv_ref[...] = jnp.full((rows, 128), -jnp.inf, jnp.float32)
            acci_ref[...] = jnp.zeros((rows, 128), jnp.int32)

        colb = jax.lax.broadcasted_iota(jnp.int32, (rows, _CHUNK), 1)
        rowb = jax.lax.broadcasted_iota(jnp.int32, (rows, _CHUNK), 0)
        linbase = ((rowb + base_ref[0]) * _VOCAB + colb
                   + j * _BLOCK_V).astype(jnp.uint32)

        def step(k, carry):
            accv, acci = carry
            off = k * _CHUNK
            lin = linbase + off.astype(jnp.uint32)
            fl = logits_ref[:, pl.ds(off, _CHUNK)] * jnp.float32(2.0)
            fl = fl + mask_ref[:, pl.ds(off, _CHUNK)]
            out_ref[:, pl.ds(off, _CHUNK)] = fl
            cand = fl + _gumbel_from_x1(lin + jnp.uint32(_KS1))
            lin_i = lin.astype(jnp.int32)
            for s in range(_NSUB):
                c = cand[:, s * 128:(s + 1) * 128]
                li = lin_i[:, s * 128:(s + 1) * 128]
                better = c > accv
                acci = jnp.where(better, li, acci)
                accv = jnp.where(better, c, accv)
            return (accv, acci)

        accv, acci = lax.fori_loop(
            0, _NCH, step, (accv_ref[...], acci_ref[...])
        )
        accv_ref[...] = accv
        acci_ref[...] = acci

        @pl.when(j == _GRID - 1)
        def _done():
            row = jax.lax.broadcasted_iota(jnp.int32, (rows, 128), 0)
            col = acci - (row + base_ref[0]) * _VOCAB
            m = jnp.max(accv, axis=1, keepdims=True)
            ids_ref[...] = jnp.min(
                jnp.where(accv == m, col, jnp.int32(2**30)),
                axis=1, keepdims=True,
            )

    return _body


def _run_shard(rows, base, logits_shard, mask2d):
    final_logits, ids = pl.pallas_call(
        _make_body(rows),
        grid_spec=pltpu.PrefetchScalarGridSpec(
            num_scalar_prefetch=1,
            grid=(_GRID,),
            in_specs=[
                pl.BlockSpec((rows, _BLOCK_V), lambda j, s: (0, j)),
                pl.BlockSpec((1, _BLOCK_V), lambda j, s: (0, j)),
            ],
            out_specs=[
                pl.BlockSpec((rows, _BLOCK_V), lambda j, s: (0, j)),
                pl.BlockSpec((rows, 1), lambda j, s: (0, 0)),
            ],
            scratch_shapes=[
                pltpu.VMEM((rows, 128), jnp.float32),
                pltpu.VMEM((rows, 128), jnp.int32),
            ],
        ),
        out_shape=[
            jax.ShapeDtypeStruct((rows, _VOCAB), jnp.float32),
            jax.ShapeDtypeStruct((rows, 1), jnp.int32),
        ],
        compiler_params=pltpu.CompilerParams(
            dimension_semantics=("arbitrary",),
        ),
    )(base, logits_shard, mask2d)
    return final_logits, ids.reshape(rows)


def kernel(logits, prediction_mask):
    mask2d = jnp.concatenate(
        [
            prediction_mask.reshape(1, _VOCAB),
            jnp.full((1, _PADV - _VOCAB), -jnp.inf, jnp.float32),
        ],
        axis=1,
    )
    devs = jax.devices()
    ndev = 2 if len(devs) >= 2 else 1
    rows = _BATCH // ndev
    if ndev == 1:
        base = jnp.zeros((1,), jnp.int32)
        return _run_shard(rows, base, logits, mask2d)

    mesh = jax.make_mesh(
        (ndev,), ("b",),
        axis_types=(jax.sharding.AxisType.Explicit,),
        devices=devs[:ndev],
    )
    logits_s = jax.device_put(
        logits, jax.sharding.NamedSharding(mesh, P("b", None))
    )
    mask_s = jax.device_put(
        mask2d, jax.sharding.NamedSharding(mesh, P(None, None))
    )

    def shard_fn(lg, mk):
        base = (jax.lax.axis_index("b") * rows).reshape(1).astype(jnp.int32)
        return _run_shard(rows, base, lg, mk)

    f = jax.shard_map(
        shard_fn,
        mesh=mesh,
        in_specs=(P("b", None), P(None, None)),
        out_specs=(P("b", None), P("b")),
        check_vma=False,
    )
    return f(logits_s, mask_s)


# speedup vs baseline: 1.4176x; 1.3634x over previous
import jax
import jax.numpy as jnp
from jax import lax
from jax.experimental import pallas as pl
from jax.experimental.pallas import tpu as pltpu

_BATCH = 32
_VOCAB = 1_000_000
_BLOCK_V = 32768
_GRID = (_VOCAB + _BLOCK_V - 1) // _BLOCK_V
_PADV = _GRID * _BLOCK_V
_NSUB = _BLOCK_V // 128

_KS1 = 42
_KS2 = 0x1BD11BDA ^ 42
_C1 = _KS2 + 1
_C2 = 2
_C3 = _KS1 + 3
_C4 = _KS2 + 4
_C5 = 5
_R0 = (13, 15, 26, 6)
_R1 = (17, 29, 16, 24)


def _rotl(x, d):
    return (x << jnp.uint32(d)) | (x >> jnp.uint32(32 - d))


def _rounds(x0, x1, rots):
    for r in rots:
        x0 = x0 + x1
        x1 = x0 ^ _rotl(x1, r)
    return x0, x1


def _bits_from_x1(x1):
    x0 = x1
    x1 = x0 ^ _rotl(x1, 13)
    x0, x1 = _rounds(x0, x1, _R0[1:])
    x0, x1 = x0 + jnp.uint32(_KS1), x1 + jnp.uint32(_C1)
    x0, x1 = _rounds(x0, x1, _R1)
    x0, x1 = x0 + jnp.uint32(_KS2), x1 + jnp.uint32(_C2)
    x0, x1 = _rounds(x0, x1, _R0)
    x0, x1 = x0, x1 + jnp.uint32(_C3)
    x0, x1 = _rounds(x0, x1, _R1)
    x0, x1 = x0 + jnp.uint32(_KS1), x1 + jnp.uint32(_C4)
    x0, x1 = _rounds(x0, x1, _R0)
    x0, x1 = x0 + jnp.uint32(_KS2), x1 + jnp.uint32(_C5)
    return x0 ^ x1


def _gumbel_from_x1(x1):
    bits = _bits_from_x1(x1)
    float_bits = (bits >> jnp.uint32(9)) | jnp.uint32(0x3F800000)
    f = lax.bitcast_convert_type(float_bits, jnp.float32) - jnp.float32(1.0)
    u = jnp.maximum(f, jnp.float32(1e-20))
    return -jnp.log(-jnp.log(u))


def _body(logits_ref, mask_ref, out_ref, ids_ref, accv_ref, acci_ref):
    j = pl.program_id(0)

    @pl.when(j == 0)
    def _init():
        accv_ref[...] = jnp.full((_BATCH, 128), -jnp.inf, jnp.float32)
        acci_ref[...] = jnp.zeros((_BATCH, 128), jnp.int32)

    colb = jax.lax.broadcasted_iota(jnp.int32, (_BATCH, _BLOCK_V), 1)
    rowb = jax.lax.broadcasted_iota(jnp.int32, (_BATCH, _BLOCK_V), 0)
    x1 = (rowb * _VOCAB + colb + (j * _BLOCK_V + _KS1)).astype(jnp.uint32)

    fl = logits_ref[...] * jnp.float32(2.0) + mask_ref[...]
    out_ref[...] = fl
    cand = fl + _gumbel_from_x1(x1)

    accv = accv_ref[...]
    acci = acci_ref[...]
    sub0 = j * _NSUB
    for s in range(_NSUB):
        c = cand[:, s * 128:(s + 1) * 128]
        better = c > accv
        acci = jnp.where(better, sub0 + s, acci)
        accv = jnp.where(better, c, accv)
    accv_ref[...] = accv
    acci_ref[...] = acci

    @pl.when(j == _GRID - 1)
    def _done():
        lane = jax.lax.broadcasted_iota(jnp.int32, (_BATCH, 128), 1)
        col = acci * 128 + lane
        m = jnp.max(accv, axis=1, keepdims=True)
        ids_ref[...] = jnp.min(
            jnp.where(accv == m, col, jnp.int32(2**30)),
            axis=1, keepdims=True,
        )


def kernel(logits, prediction_mask):
    mask2d = jnp.concatenate(
        [
            prediction_mask.reshape(1, _VOCAB),
            jnp.full((1, _PADV - _VOCAB), -jnp.inf, jnp.float32),
        ],
        axis=1,
    )
    final_logits, ids = pl.pallas_call(
        _body,
        grid=(_GRID,),
        in_specs=[
            pl.BlockSpec((_BATCH, _BLOCK_V), lambda j: (0, j)),
            pl.BlockSpec((1, _BLOCK_V), lambda j: (0, j)),
        ],
        out_specs=[
            pl.BlockSpec((_BATCH, _BLOCK_V), lambda j: (0, j)),
            pl.BlockSpec((_BATCH, 1), lambda j: (0, 0)),
        ],
        out_shape=[
            jax.ShapeDtypeStruct((_BATCH, _VOCAB), jnp.float32),
            jax.ShapeDtypeStruct((_BATCH, 1), jnp.int32),
        ],
        scratch_shapes=[
            pltpu.VMEM((_BATCH, 128), jnp.float32),
            pltpu.VMEM((_BATCH, 128), jnp.int32),
        ],
        compiler_params=pltpu.CompilerParams(
            dimension_semantics=("arbitrary",),
        ),
    )(logits, mask2d)
    return final_logits, ids.reshape(_BATCH)
